# baseline (speedup 1.0000x reference)
import jax
import jax.numpy as jnp
from jax.experimental import pallas as pl
from jax.experimental.pallas import tpu as pltpu

_DEG = 3
_NC = 128
_NSEG = _NC - _DEG
_N = 1024
_TILE = 128
_PREC = jax.lax.Precision.HIGHEST


def _span_basis(t2):
    kj = jax.lax.broadcasted_iota(jnp.int32, (_NC, _N), 0).astype(jnp.float32)
    pred = (t2 - kj / float(_NSEG)) > 1e-8
    cnt = jnp.sum(pred.astype(jnp.int32), axis=0, keepdims=True)
    span = jnp.maximum(cnt - 1, 0) + _DEG
    span_f = span.astype(jnp.float32)

    N = [jnp.ones_like(t2), None, None, None]
    for k in range(1, _DEG + 1):
        saved = jnp.zeros_like(t2)
        for r in range(k):
            V1 = jnp.clip((span_f + float(r - 2)) / float(_NSEG), 0.0, 1.0)
            V2 = jnp.clip((span_f + float(r - k - 2)) / float(_NSEG), 0.0, 1.0)
            denom = (V1 - t2) + (t2 - V2)
            temp = jnp.where(denom == 0.0, jnp.full_like(t2, 0.0001),
                             N[r] / denom)
            N[r] = saved + (V1 - t2) * temp
            saved = (t2 - V2) * temp
        N[k] = saved
    return span, N


def _basis_mat_t(span, N):
    m = jax.lax.broadcasted_iota(jnp.int32, (_NC, _N), 0)
    base = span - _DEG
    acc = jnp.zeros((_NC, _N), jnp.float32)
    for l in range(_DEG + 1):
        acc = acc + jnp.where(m == base + l, N[l], 0.0)
    return acc


def _split(x):
    hi = x.astype(jnp.bfloat16)
    lo = (x - hi.astype(jnp.float32)).astype(jnp.bfloat16)
    return hi, lo


def _dot(a, b):
    return jax.lax.dot_general(a, b, (((0,), (0,)), ((), ())),
                               preferred_element_type=jnp.float32)


def _body(cp_ref, w_ref, u_ref, v_ref, out_ref, buthi_ref, thi_ref, tlo_ref):
    i = pl.program_id(0)

    @pl.when(i == 0)
    def _setup():
        q = jax.lax.broadcasted_iota(jnp.int32, (3 * _NC, _NC), 0)
        n3 = 3 * jax.lax.broadcasted_iota(jnp.int32, (3 * _NC, _NC), 1)
        cphi, cplo = _split(cp_ref[...])
        wmat = w_ref[...]

        sv, Nv = _span_basis(v_ref[...])
        bvt = _basis_mat_t(sv, Nv)
        for c in range(4):
            if c == 3:
                ctrl_c = wmat
            else:
                sel = jnp.where(q == n3 + c, 1.0, 0.0).astype(jnp.bfloat16)
                cp_c = (jnp.dot(cphi, sel, preferred_element_type=jnp.float32)
                        + jnp.dot(cplo, sel, preferred_element_type=jnp.float32))
                ctrl_c = cp_c * wmat
            tc = jnp.dot(ctrl_c, bvt, preferred_element_type=jnp.float32,
                         precision=_PREC)
            thi_ref[c], tlo_ref[c] = _split(tc)
        su, Nu = _span_basis(u_ref[...])
        but = _basis_mat_t(su, Nu)
        buthi_ref[...] = but.astype(jnp.bfloat16)

    bhi = buthi_ref[:, pl.ds(i * _TILE, _TILE)]
    r = [_dot(bhi, thi_ref[c]) + _dot(bhi, tlo_ref[c]) for c in range(4)]
    winv = 1.0 / r[3]
    for c in range(3):
        out_ref[c] = r[c] * winv


def kernel(control_pts, weights, u_spline_space, v_spline_space):
    cp2 = control_pts.reshape(_NC, 3 * _NC)
    w2 = weights.reshape(_NC, _NC)
    u2 = jnp.sort(u_spline_space)[None, :]
    v2 = v_spline_space[None, :]

    out2d = pl.pallas_call(
        _body,
        grid=(_N // _TILE,),
        in_specs=[
            pl.BlockSpec((_NC, 3 * _NC), lambda i: (0, 0)),
            pl.BlockSpec((_NC, _NC), lambda i: (0, 0)),
            pl.BlockSpec((1, _N), lambda i: (0, 0)),
            pl.BlockSpec((1, _N), lambda i: (0, 0)),
        ],
        out_specs=pl.BlockSpec((3, _TILE, _N), lambda i: (0, i, 0)),
        out_shape=jax.ShapeDtypeStruct((3, _N, _N), jnp.float32),
        scratch_shapes=[
            pltpu.VMEM((_NC, _N), jnp.bfloat16),
            pltpu.VMEM((4, _NC, _N), jnp.bfloat16),
            pltpu.VMEM((4, _NC, _N), jnp.bfloat16),
        ],
    )(cp2, w2, u2, v2)
    return jnp.transpose(out2d, (1, 2, 0))[None]


# speedup vs baseline: 1.1927x; 1.1927x over previous
import jax
import jax.numpy as jnp
from jax.experimental import pallas as pl
from jax.experimental.pallas import tpu as pltpu

_DEG = 3
_NC = 128
_NSEG = _NC - _DEG
_N = 1024
_TILE = 256
_PREC = jax.lax.Precision.HIGHEST


def _span_basis(t2):
    w = t2.shape[1]
    kj = jax.lax.broadcasted_iota(jnp.int32, (_NC, w), 0).astype(jnp.float32)
    pred = (t2 - kj / float(_NSEG)) > 1e-8
    cnt = jnp.sum(pred.astype(jnp.int32), axis=0, keepdims=True)
    span = jnp.maximum(cnt - 1, 0) + _DEG
    span_f = span.astype(jnp.float32)

    N = [jnp.ones_like(t2), None, None, None]
    for k in range(1, _DEG + 1):
        saved = jnp.zeros_like(t2)
        for r in range(k):
            V1 = jnp.clip((span_f + float(r - 2)) / float(_NSEG), 0.0, 1.0)
            V2 = jnp.clip((span_f + float(r - k - 2)) / float(_NSEG), 0.0, 1.0)
            denom = (V1 - t2) + (t2 - V2)
            temp = jnp.where(denom == 0.0, jnp.full_like(t2, 0.0001),
                             N[r] / denom)
            N[r] = saved + (V1 - t2) * temp
            saved = (t2 - V2) * temp
        N[k] = saved
    return span, N


def _basis_mat_t(span, N):
    w = span.shape[1]
    m = jax.lax.broadcasted_iota(jnp.int32, (_NC, w), 0)
    base = span - _DEG
    acc = jnp.zeros((_NC, w), jnp.float32)
    for l in range(_DEG + 1):
        acc = acc + jnp.where(m == base + l, N[l], 0.0)
    return acc


def _split(x):
    hi = x.astype(jnp.bfloat16)
    lo = (x - hi.astype(jnp.float32)).astype(jnp.bfloat16)
    return hi, lo


def _dot(a, b):
    return jax.lax.dot_general(a, b, (((0,), (0,)), ((), ())),
                               preferred_element_type=jnp.float32)


def _body(cp_ref, w_ref, ur_ref, uc_ref, v_ref, out_ref,
          rank_ref, thi_ref, tlo_ref):
    i = pl.program_id(0)

    @pl.when(i == 0)
    def _setup():
        q = jax.lax.broadcasted_iota(jnp.int32, (3 * _NC, _NC), 0)
        n3 = 3 * jax.lax.broadcasted_iota(jnp.int32, (3 * _NC, _NC), 1)
        cphi, cplo = _split(cp_ref[...])
        wmat = w_ref[...]

        sv, Nv = _span_basis(v_ref[...])
        bvt = _basis_mat_t(sv, Nv)
        for c in range(4):
            if c == 3:
                ctrl_c = wmat
            else:
                sel = jnp.where(q == n3 + c, 1.0, 0.0).astype(jnp.bfloat16)
                cp_c = (jnp.dot(cphi, sel, preferred_element_type=jnp.float32)
                        + jnp.dot(cplo, sel, preferred_element_type=jnp.float32))
                ctrl_c = cp_c * wmat
            tc = jnp.dot(ctrl_c, bvt, preferred_element_type=jnp.float32,
                         precision=_PREC)
            thi_ref[c], tlo_ref[c] = _split(tc)

        urow = ur_ref[...]
        ucol = uc_ref[...]
        k_io = jax.lax.broadcasted_iota(jnp.int32, (_N, _N), 1)
        j_io = jax.lax.broadcasted_iota(jnp.int32, (_N, _N), 0)
        lt = urow < ucol
        tie = (urow == ucol) & (k_io < j_io)
        rank_ref[...] = jnp.sum((lt | tie).astype(jnp.int32), axis=1,
                                keepdims=True)

    p_io = (jax.lax.broadcasted_iota(jnp.int32, (_N, _TILE), 1) + i * _TILE)
    mask = rank_ref[...] == p_io
    u_t = jnp.sum(jnp.where(mask, uc_ref[...], 0.0), axis=0, keepdims=True)
    su, Nu = _span_basis(u_t)
    bhi = _basis_mat_t(su, Nu).astype(jnp.bfloat16)

    r = [_dot(bhi, thi_ref[c]) + _dot(bhi, tlo_ref[c]) for c in range(4)]
    winv = 1.0 / r[3]
    for c in range(3):
        out_ref[c] = r[c] * winv


def kernel(control_pts, weights, u_spline_space, v_spline_space):
    cp2 = control_pts.reshape(_NC, 3 * _NC)
    w2 = weights.reshape(_NC, _NC)
    ur = u_spline_space[None, :]
    uc = u_spline_space[:, None]
    v2 = v_spline_space[None, :]

    out2d = pl.pallas_call(
        _body,
        grid=(_N // _TILE,),
        in_specs=[
            pl.BlockSpec((_NC, 3 * _NC), lambda i: (0, 0)),
            pl.BlockSpec((_NC, _NC), lambda i: (0, 0)),
            pl.BlockSpec((1, _N), lambda i: (0, 0)),
            pl.BlockSpec((_N, 1), lambda i: (0, 0)),
            pl.BlockSpec((1, _N), lambda i: (0, 0)),
        ],
        out_specs=pl.BlockSpec((3, _TILE, _N), lambda i: (0, i, 0)),
        out_shape=jax.ShapeDtypeStruct((3, _N, _N), jnp.float32),
        scratch_shapes=[
            pltpu.VMEM((_N, 1), jnp.int32),
            pltpu.VMEM((4, _NC, _N), jnp.bfloat16),
            pltpu.VMEM((4, _NC, _N), jnp.bfloat16),
        ],
    )(cp2, w2, ur, uc, v2)
    return jnp.transpose(out2d, (1, 2, 0))[None]
